# baseline (device time: 127005 ns/iter reference)
import jax
import jax.numpy as jnp
from jax import lax
from jax.experimental import pallas as pl
from jax.experimental.pallas import tpu as pltpu

N_DEV = 4
SQ = 1024
SKV = 1024
HQ = 8
DH = 128
D_MODEL = 1024
SCALE = 0.08838834764831843
BLOCK = 64


def _body(x_ref, wq_ref, k_ref, v_ref, wo_ref, out_ref,
          ctx_ref, comm_ref, send_sems, recv_sems):
    my = lax.axis_index("i")
    left = (my + N_DEV - 1) % N_DEV
    right = (my + 1) % N_DEV

    barrier_sem = pltpu.get_barrier_semaphore()
    for nbr in (left, right):
        pl.semaphore_signal(barrier_sem, inc=1, device_id=(nbr,),
                            device_id_type=pl.DeviceIdType.MESH)
    pl.semaphore_wait(barrier_sem, 2)

    q = jnp.dot(x_ref[...], wq_ref[...],
                preferred_element_type=jnp.float32).astype(jnp.bfloat16)

    r = lax.broadcasted_iota(jnp.int32, (SQ, SKV), 0)
    c = lax.broadcasted_iota(jnp.int32, (SQ, SKV), 1)
    bias = jnp.where((r // BLOCK) % 4 == (c // BLOCK) % 4,
                     jnp.float32(0.0), jnp.float32(-1e9))

    for h in range(HQ):
        qh = q[:, h * DH:(h + 1) * DH]
        s = lax.dot_general(qh, k_ref[h], (((1,), (1,)), ((), ())),
                            preferred_element_type=jnp.float32)
        s = s * SCALE + bias
        m = jnp.max(s, axis=-1, keepdims=True)
        w = jnp.exp(s - m)
        w = w / jnp.sum(w, axis=-1, keepdims=True)
        ctx_h = jnp.dot(w.astype(jnp.bfloat16), v_ref[h],
                        preferred_element_type=jnp.float32)
        ctx_ref[:, h * DH:(h + 1) * DH] = ctx_h.astype(jnp.bfloat16)

    partial = jnp.dot(ctx_ref[...], wo_ref[...],
                      preferred_element_type=jnp.float32)
    out_ref[0] = partial
    comm_ref[3] = partial.astype(jnp.bfloat16)

    for h in range(N_DEV - 1):
        src_slot = 3 if h == 0 else h - 1
        rdma = pltpu.make_async_remote_copy(
            src_ref=comm_ref.at[src_slot],
            dst_ref=comm_ref.at[h],
            send_sem=send_sems.at[h],
            recv_sem=recv_sems.at[h],
            device_id=(right,),
            device_id_type=pl.DeviceIdType.MESH,
        )
        rdma.start()
        rdma.wait()
        out_ref[0] += comm_ref[h].astype(jnp.float32)


def kernel(x, Wq, K_ext, V_ext, Wo):
    my = lax.axis_index("i")
    xb = x[0].astype(jnp.bfloat16)
    wq = Wq.astype(jnp.bfloat16)
    wo = Wo.astype(jnp.bfloat16)
    k = lax.dynamic_slice_in_dim(K_ext[0], my * HQ, HQ, axis=1)
    v = lax.dynamic_slice_in_dim(V_ext[0], my * HQ, HQ, axis=1)
    k = jnp.transpose(k, (1, 0, 2)).astype(jnp.bfloat16)
    v = jnp.transpose(v, (1, 0, 2)).astype(jnp.bfloat16)

    return pl.pallas_call(
        _body,
        out_shape=jax.ShapeDtypeStruct((1, SQ, D_MODEL), jnp.float32),
        in_specs=[pl.BlockSpec(memory_space=pltpu.VMEM)] * 5,
        out_specs=pl.BlockSpec(memory_space=pltpu.VMEM),
        scratch_shapes=[
            pltpu.VMEM((SQ, HQ * DH), jnp.bfloat16),
            pltpu.VMEM((N_DEV, SQ, D_MODEL), jnp.bfloat16),
            pltpu.SemaphoreType.DMA((N_DEV - 1,)),
            pltpu.SemaphoreType.DMA((N_DEV - 1,)),
        ],
        compiler_params=pltpu.CompilerParams(
            collective_id=0,
            vmem_limit_bytes=120 * 1024 * 1024,
        ),
    )(xb, wq, k, v, wo)


# device time: 83576 ns/iter; 1.5196x vs baseline; 1.5196x over previous
import jax
import jax.numpy as jnp
from jax import lax
from jax.experimental import pallas as pl
from jax.experimental.pallas import tpu as pltpu

N_DEV = 4
SQ = 1024
SKV = 1024
HQ = 8
DH = 128
D_MODEL = 1024
SCALE = 0.08838834764831843
BLOCK = 64
CHUNK = SQ // N_DEV


def _body(x_ref, wq_ref, k_ref, v_ref, wo_ref, out_ref,
          q_ref, ctx_ref, rs_send, rs_recv, ag_send, ag_recv,
          rs_send_sems, rs_recv_sems, ag_send_sems, ag_recv_sems):
    my = lax.axis_index("i")

    barrier_sem = pltpu.get_barrier_semaphore()
    for d in range(1, N_DEV):
        pl.semaphore_signal(barrier_sem, inc=1,
                            device_id=((my + d) % N_DEV,),
                            device_id_type=pl.DeviceIdType.MESH)
    pl.semaphore_wait(barrier_sem, N_DEV - 1)

    q_ref[...] = jnp.dot(x_ref[...], wq_ref[...],
                         preferred_element_type=jnp.float32
                         ).astype(jnp.bfloat16)

    r = lax.broadcasted_iota(jnp.int32, (CHUNK, SKV), 0)
    c_ = lax.broadcasted_iota(jnp.int32, (CHUNK, SKV), 1)
    bias = jnp.where((r // BLOCK) % 4 == (c_ // BLOCK) % 4,
                     jnp.float32(0.0), jnp.float32(-1e9))

    for c in range(N_DEV):
        rows = slice(c * CHUNK, (c + 1) * CHUNK)
        for h in range(HQ):
            qh = q_ref[rows, h * DH:(h + 1) * DH]
            s = lax.dot_general(qh, k_ref[h], (((1,), (1,)), ((), ())),
                                preferred_element_type=jnp.float32)
            s = s * SCALE + bias
            m = jnp.max(s, axis=-1, keepdims=True)
            w = jnp.exp(s - m)
            w = w / jnp.sum(w, axis=-1, keepdims=True)
            ctx_h = jnp.dot(w.astype(jnp.bfloat16), v_ref[h],
                            preferred_element_type=jnp.float32)
            ctx_ref[:, h * DH:(h + 1) * DH] = ctx_h.astype(jnp.bfloat16)

        partial = jnp.dot(ctx_ref[...], wo_ref[...],
                          preferred_element_type=jnp.float32)
        out_ref[rows, :] = partial
        rs_send[c] = partial.astype(jnp.bfloat16)

        @pl.when(c != my)
        def _():
            pltpu.make_async_remote_copy(
                src_ref=rs_send.at[c], dst_ref=rs_recv.at[my],
                send_sem=rs_send_sems.at[c], recv_sem=rs_recv_sems.at[my],
                device_id=(c,), device_id_type=pl.DeviceIdType.MESH,
            ).start()

    for s_ in range(N_DEV):
        @pl.when(s_ != my)
        def _():
            pltpu.make_async_remote_copy(
                src_ref=rs_send.at[s_], dst_ref=rs_recv.at[s_],
                send_sem=rs_send_sems.at[s_], recv_sem=rs_recv_sems.at[s_],
                device_id=(s_,), device_id_type=pl.DeviceIdType.MESH,
            ).wait_recv()

    my_rows = pl.ds(my * CHUNK, CHUNK)
    own = out_ref[my_rows, :]
    for s_ in range(N_DEV):
        own = own + jnp.where(s_ == my, jnp.float32(0.0),
                              rs_recv[s_].astype(jnp.float32))
    out_ref[my_rows, :] = own
    ag_send[...] = own.astype(jnp.bfloat16)

    for c in range(N_DEV):
        @pl.when(c != my)
        def _():
            pltpu.make_async_remote_copy(
                src_ref=ag_send, dst_ref=ag_recv.at[my],
                send_sem=ag_send_sems.at[c], recv_sem=ag_recv_sems.at[my],
                device_id=(c,), device_id_type=pl.DeviceIdType.MESH,
            ).start()

    for c in range(N_DEV):
        @pl.when(c != my)
        def _():
            pltpu.make_async_remote_copy(
                src_ref=rs_send.at[c], dst_ref=rs_recv.at[my],
                send_sem=rs_send_sems.at[c], recv_sem=rs_recv_sems.at[my],
                device_id=(c,), device_id_type=pl.DeviceIdType.MESH,
            ).wait_send()

    for s_ in range(N_DEV):
        @pl.when(s_ != my)
        def _():
            pltpu.make_async_remote_copy(
                src_ref=ag_send, dst_ref=ag_recv.at[s_],
                send_sem=ag_send_sems.at[s_], recv_sem=ag_recv_sems.at[s_],
                device_id=(s_,), device_id_type=pl.DeviceIdType.MESH,
            ).wait_recv()
            out_ref[s_ * CHUNK:(s_ + 1) * CHUNK, :] = (
                ag_recv[s_].astype(jnp.float32))

    for c in range(N_DEV):
        @pl.when(c != my)
        def _():
            pltpu.make_async_remote_copy(
                src_ref=ag_send, dst_ref=ag_recv.at[my],
                send_sem=ag_send_sems.at[c], recv_sem=ag_recv_sems.at[my],
                device_id=(c,), device_id_type=pl.DeviceIdType.MESH,
            ).wait_send()


def kernel(x, Wq, K_ext, V_ext, Wo):
    my = lax.axis_index("i")
    xb = x[0].astype(jnp.bfloat16)
    wq = Wq.astype(jnp.bfloat16)
    wo = Wo.astype(jnp.bfloat16)
    k = lax.dynamic_slice_in_dim(K_ext[0], my * HQ, HQ, axis=1)
    v = lax.dynamic_slice_in_dim(V_ext[0], my * HQ, HQ, axis=1)
    k = jnp.transpose(k, (1, 0, 2)).astype(jnp.bfloat16)
    v = jnp.transpose(v, (1, 0, 2)).astype(jnp.bfloat16)

    out = pl.pallas_call(
        _body,
        out_shape=jax.ShapeDtypeStruct((SQ, D_MODEL), jnp.float32),
        in_specs=[pl.BlockSpec(memory_space=pltpu.VMEM)] * 5,
        out_specs=pl.BlockSpec(memory_space=pltpu.VMEM),
        scratch_shapes=[
            pltpu.VMEM((SQ, HQ * DH), jnp.bfloat16),
            pltpu.VMEM((CHUNK, HQ * DH), jnp.bfloat16),
            pltpu.VMEM((N_DEV, CHUNK, D_MODEL), jnp.bfloat16),
            pltpu.VMEM((N_DEV, CHUNK, D_MODEL), jnp.bfloat16),
            pltpu.VMEM((CHUNK, D_MODEL), jnp.bfloat16),
            pltpu.VMEM((N_DEV, CHUNK, D_MODEL), jnp.bfloat16),
            pltpu.SemaphoreType.DMA((N_DEV,)),
            pltpu.SemaphoreType.DMA((N_DEV,)),
            pltpu.SemaphoreType.DMA((N_DEV,)),
            pltpu.SemaphoreType.DMA((N_DEV,)),
        ],
        compiler_params=pltpu.CompilerParams(
            collective_id=0,
            vmem_limit_bytes=120 * 1024 * 1024,
        ),
    )(xb, wq, k, v, wo)
    return out[None]


# device time: 67348 ns/iter; 1.8858x vs baseline; 1.2410x over previous
import jax
import jax.numpy as jnp
from jax import lax
from jax.experimental import pallas as pl
from jax.experimental.pallas import tpu as pltpu

N_DEV = 4
SQ = 1024
SKV = 1024
HQ = 8
DH = 128
D_MODEL = 1024
SCALE = 0.08838834764831843
BLOCK = 64
NSTRIDE = 4
NREP = 4
GROUP = NREP * BLOCK
CHUNK = SQ // N_DEV


def _body(x_ref, wq_ref, k_ref, v_ref, wo_ref, out_ref,
          q_ref, wob_ref, ctx_ref, chnk_ref,
          rs_send, rs_recv, ag_send, ag_recv,
          rs_send_sems, rs_recv_sems, ag_send_sems, ag_recv_sems):
    my = lax.axis_index("i")

    barrier_sem = pltpu.get_barrier_semaphore()
    for d in range(1, N_DEV):
        pl.semaphore_signal(barrier_sem, inc=1,
                            device_id=((my + d) % N_DEV,),
                            device_id_type=pl.DeviceIdType.MESH)
    pl.semaphore_wait(barrier_sem, N_DEV - 1)

    q_ref[...] = jnp.dot(x_ref[...].astype(jnp.bfloat16),
                         wq_ref[...].astype(jnp.bfloat16),
                         preferred_element_type=jnp.float32
                         ).astype(jnp.bfloat16)
    wob_ref[...] = wo_ref[...].astype(jnp.bfloat16)

    for h in range(HQ):
        for s in range(NSTRIDE):
            rows = slice(s * GROUP, (s + 1) * GROUP)
            qs = q_ref[rows, h * DH:(h + 1) * DH]
            kb = k_ref[h, s].astype(jnp.bfloat16)
            vb = v_ref[h, s].astype(jnp.bfloat16)
            sc = lax.dot_general(qs, kb, (((1,), (1,)), ((), ())),
                                 preferred_element_type=jnp.float32)
            w = jnp.exp(sc * SCALE)
            rsum = 1.0 / jnp.sum(w, axis=-1, keepdims=True)
            ctx = jnp.dot(w.astype(jnp.bfloat16), vb,
                          preferred_element_type=jnp.float32)
            ctx_ref[rows, h * DH:(h + 1) * DH] = (
                (ctx * rsum).astype(jnp.bfloat16))

    for c in range(N_DEV):
        for s in range(NSTRIDE):
            chnk_ref[s * BLOCK:(s + 1) * BLOCK, :] = (
                ctx_ref[s * GROUP + c * BLOCK:
                        s * GROUP + (c + 1) * BLOCK, :])
        partial = jnp.dot(chnk_ref[...], wob_ref[...],
                          preferred_element_type=jnp.float32)
        out_ref[c * CHUNK:(c + 1) * CHUNK, :] = partial
        rs_send[c] = partial.astype(jnp.bfloat16)

        @pl.when(c != my)
        def _():
            pltpu.make_async_remote_copy(
                src_ref=rs_send.at[c], dst_ref=rs_recv.at[my],
                send_sem=rs_send_sems.at[c], recv_sem=rs_recv_sems.at[my],
                device_id=(c,), device_id_type=pl.DeviceIdType.MESH,
            ).start()

    for s_ in range(N_DEV):
        @pl.when(s_ != my)
        def _():
            pltpu.make_async_remote_copy(
                src_ref=rs_send.at[s_], dst_ref=rs_recv.at[s_],
                send_sem=rs_send_sems.at[s_], recv_sem=rs_recv_sems.at[s_],
                device_id=(s_,), device_id_type=pl.DeviceIdType.MESH,
            ).wait_recv()

    my_rows = pl.ds(my * CHUNK, CHUNK)
    own = out_ref[my_rows, :]
    for s_ in range(N_DEV):
        own = own + jnp.where(s_ == my, jnp.float32(0.0),
                              rs_recv[s_].astype(jnp.float32))
    out_ref[my_rows, :] = own
    ag_send[...] = own.astype(jnp.bfloat16)

    for c in range(N_DEV):
        @pl.when(c != my)
        def _():
            pltpu.make_async_remote_copy(
                src_ref=ag_send, dst_ref=ag_recv.at[my],
                send_sem=ag_send_sems.at[c], recv_sem=ag_recv_sems.at[my],
                device_id=(c,), device_id_type=pl.DeviceIdType.MESH,
            ).start()

    for c in range(N_DEV):
        @pl.when(c != my)
        def _():
            pltpu.make_async_remote_copy(
                src_ref=rs_send.at[c], dst_ref=rs_recv.at[my],
                send_sem=rs_send_sems.at[c], recv_sem=rs_recv_sems.at[my],
                device_id=(c,), device_id_type=pl.DeviceIdType.MESH,
            ).wait_send()

    for s_ in range(N_DEV):
        @pl.when(s_ != my)
        def _():
            pltpu.make_async_remote_copy(
                src_ref=ag_send, dst_ref=ag_recv.at[s_],
                send_sem=ag_send_sems.at[s_], recv_sem=ag_recv_sems.at[s_],
                device_id=(s_,), device_id_type=pl.DeviceIdType.MESH,
            ).wait_recv()
            out_ref[s_ * CHUNK:(s_ + 1) * CHUNK, :] = (
                ag_recv[s_].astype(jnp.float32))

    for c in range(N_DEV):
        @pl.when(c != my)
        def _():
            pltpu.make_async_remote_copy(
                src_ref=ag_send, dst_ref=ag_recv.at[my],
                send_sem=ag_send_sems.at[c], recv_sem=ag_recv_sems.at[my],
                device_id=(c,), device_id_type=pl.DeviceIdType.MESH,
            ).wait_send()


def kernel(x, Wq, K_ext, V_ext, Wo):
    my = lax.axis_index("i")
    xp = x[0].reshape(NREP, NSTRIDE, BLOCK, D_MODEL)
    xp = jnp.transpose(xp, (1, 0, 2, 3)).reshape(SQ, D_MODEL)

    def group_kv(t):
        g = lax.dynamic_slice_in_dim(t[0], my * HQ, HQ, axis=1)
        g = g.reshape(NREP, NSTRIDE, BLOCK, HQ, DH)
        return jnp.transpose(g, (3, 1, 0, 2, 4)).reshape(
            HQ, NSTRIDE, GROUP, DH)

    k = group_kv(K_ext)
    v = group_kv(V_ext)

    out = pl.pallas_call(
        _body,
        out_shape=jax.ShapeDtypeStruct((SQ, D_MODEL), jnp.float32),
        in_specs=[pl.BlockSpec(memory_space=pltpu.VMEM)] * 5,
        out_specs=pl.BlockSpec(memory_space=pltpu.VMEM),
        scratch_shapes=[
            pltpu.VMEM((SQ, HQ * DH), jnp.bfloat16),
            pltpu.VMEM((HQ * DH, D_MODEL), jnp.bfloat16),
            pltpu.VMEM((SQ, HQ * DH), jnp.bfloat16),
            pltpu.VMEM((CHUNK, HQ * DH), jnp.bfloat16),
            pltpu.VMEM((N_DEV, CHUNK, D_MODEL), jnp.bfloat16),
            pltpu.VMEM((N_DEV, CHUNK, D_MODEL), jnp.bfloat16),
            pltpu.VMEM((CHUNK, D_MODEL), jnp.bfloat16),
            pltpu.VMEM((N_DEV, CHUNK, D_MODEL), jnp.bfloat16),
            pltpu.SemaphoreType.DMA((N_DEV,)),
            pltpu.SemaphoreType.DMA((N_DEV,)),
            pltpu.SemaphoreType.DMA((N_DEV,)),
            pltpu.SemaphoreType.DMA((N_DEV,)),
        ],
        compiler_params=pltpu.CompilerParams(
            collective_id=0,
            vmem_limit_bytes=120 * 1024 * 1024,
        ),
    )(xp, Wq, k, v, Wo)
    return out[None]
